# baseline (speedup 1.0000x reference)
import functools

import jax
import jax.numpy as jnp
from jax import lax
from jax.experimental import pallas as pl
from jax.experimental.pallas import tpu as pltpu
from jax.experimental.pallas import tpu_sc as plsc

VOCAB_SIZE = 1000000
EMB_SIZE = 64
N = 16384

NUM_CORES = 2
NUM_SUBCORES = 16
NUM_WORKERS = NUM_CORES * NUM_SUBCORES
B_PER_W = N // NUM_WORKERS
PACK = 2
PACKED_ROWS = VOCAB_SIZE // PACK
PASS_IDS = 128
N_PASS = B_PER_W // PASS_IDS


@functools.partial(
    pl.kernel,
    out_type=jax.ShapeDtypeStruct((N, 1, EMB_SIZE), jnp.float32),
    mesh=plsc.VectorSubcoreMesh(core_axis_name="c", subcore_axis_name="s"),
    scratch_types=[
        pltpu.VMEM((B_PER_W,), jnp.int32),
        pltpu.VMEM((B_PER_W,), jnp.int32),
        pltpu.VMEM((2, PASS_IDS, PACK * EMB_SIZE), jnp.float32),
        pltpu.VMEM((B_PER_W, 1, EMB_SIZE), jnp.float32),
        pltpu.SemaphoreType.DMA,
        pltpu.SemaphoreType.DMA,
    ],
)
def _gather_kernel(ids_hbm, table_hbm, out_hbm, idx_v, blk_v, g_v, rows_v,
                   sem_g, sem_w):
    wid = lax.axis_index("s") * NUM_CORES + lax.axis_index("c")
    base = wid * B_PER_W
    pltpu.sync_copy(ids_hbm.at[wid], idx_v)

    def blk_body(g, carry):
        ids16 = idx_v[pl.ds(g * 16, 16)]
        blk_v[pl.ds(g * 16, 16)] = lax.shift_right_logical(ids16, 1)
        return carry

    lax.fori_loop(0, B_PER_W // 16, blk_body, 0)

    def fire(p, buf):
        return pltpu.async_copy(
            table_hbm.at[blk_v.at[pl.ds(p * PASS_IDS, PASS_IDS)]],
            g_v.at[buf],
            sem_g,
        )

    def select(p, buf):
        def sel_body(g, carry):
            ids16 = idx_v[pl.ds(p * PASS_IDS + g * 16, 16)]
            for j in range(16):
                rid = ids16[j]
                half = lax.mul(lax.rem(rid, PACK), EMB_SIZE)
                n_loc = g * 16 + j
                for c in range(EMB_SIZE // 16):
                    rows_v[p * PASS_IDS + n_loc, 0, pl.ds(c * 16, 16)] = (
                        g_v[buf, n_loc, pl.ds(half + c * 16, 16)]
                    )
            return carry

        lax.fori_loop(0, PASS_IDS // 16, sel_body, 0)

    writes = []
    pending = fire(0, 0)
    for p in range(N_PASS):
        buf = p % 2
        pending.wait()
        if p + 1 < N_PASS:
            pending = fire(p + 1, 1 - buf)
        select(p, buf)
        writes.append(
            pltpu.async_copy(
                rows_v.at[pl.ds(p * PASS_IDS, PASS_IDS)],
                out_hbm.at[pl.ds(base + p * PASS_IDS, PASS_IDS)],
                sem_w,
            )
        )
    for w in writes:
        w.wait()


def kernel(vocab_ids, table):
    ids2d = vocab_ids.reshape(NUM_WORKERS, B_PER_W)
    table2d = table.reshape(PACKED_ROWS, PACK * EMB_SIZE)
    return _gather_kernel(ids2d, table2d)


# speedup vs baseline: 1.0241x; 1.0241x over previous
import functools

import jax
import jax.numpy as jnp
from jax import lax
from jax.experimental import pallas as pl
from jax.experimental.pallas import tpu as pltpu
from jax.experimental.pallas import tpu_sc as plsc

VOCAB_SIZE = 1000000
EMB_SIZE = 64
N = 16384

NUM_CORES = 2
NUM_SUBCORES = 16
NUM_WORKERS = NUM_CORES * NUM_SUBCORES
B_PER_W = N // NUM_WORKERS


@functools.partial(
    pl.kernel,
    out_type=jax.ShapeDtypeStruct((N, 1, EMB_SIZE), jnp.float32),
    mesh=plsc.VectorSubcoreMesh(core_axis_name="c", subcore_axis_name="s"),
    scratch_types=[
        pltpu.VMEM((B_PER_W,), jnp.int32),
        pltpu.SemaphoreType.DMA,
    ],
)
def _gather_kernel(ids_hbm, table_hbm, out_hbm, idx_v, sem):
    wid = lax.axis_index("s") * NUM_CORES + lax.axis_index("c")
    base = wid * B_PER_W
    pltpu.sync_copy(ids_hbm.at[wid], idx_v)

    n_groups = B_PER_W // 16

    def body(g, carry):
        @pl.when(g < n_groups)
        def _issue():
            ids16 = idx_v[pl.ds(g * 16, 16)]
            for j in range(16):
                rid = ids16[j]
                pltpu.async_copy(
                    table_hbm.at[rid], out_hbm.at[base + g * 16 + j, 0], sem
                )

        @pl.when(g > 0)
        def _drain():
            for j in range(16):
                pltpu.make_async_copy(
                    table_hbm.at[0], out_hbm.at[base, 0], sem
                ).wait()

        return carry

    lax.fori_loop(0, n_groups + 1, body, 0)


def kernel(vocab_ids, table):
    ids2d = vocab_ids.reshape(NUM_WORKERS, B_PER_W)
    return _gather_kernel(ids2d, table)
